# baseline (speedup 1.0000x reference)
import jax
import jax.numpy as jnp
from jax.experimental import pallas as pl
from jax.experimental.pallas import tpu as pltpu

_BM = 256


def _mm(a_ref, v_ref, o_ref, v_s):
    b, _, f = v_ref.shape

    @pl.when(pl.program_id(0) == 0)
    def _unpack():
        for j in range(b):
            v_s[:, j * f:(j + 1) * f] = v_ref[j]

    res = jnp.dot(a_ref[...], v_s[...], preferred_element_type=jnp.float32)
    for j in range(b):
        o_ref[j] = res[:, j * f:(j + 1) * f]


def kernel(matrix, matrix_batch):
    m, k = matrix.shape
    b, _, f = matrix_batch.shape

    return pl.pallas_call(
        _mm,
        grid=(m // _BM,),
        in_specs=[
            pl.BlockSpec((_BM, k), lambda i: (i, 0)),
            pl.BlockSpec((b, k, f), lambda i: (0, 0, 0)),
        ],
        out_specs=pl.BlockSpec((b, _BM, f), lambda i: (0, i, 0)),
        out_shape=jax.ShapeDtypeStruct((b, m, f), jnp.float32),
        scratch_shapes=[pltpu.VMEM((k, b * f), jnp.float32)],
    )(matrix, matrix_batch)


# speedup vs baseline: 1.0427x; 1.0427x over previous
import jax
import jax.numpy as jnp
from jax.experimental import pallas as pl
from jax.experimental.pallas import tpu as pltpu

_BM = 256


def _mm(a_ref, v_ref, o_ref, v_s):
    b, _, f = v_ref.shape

    @pl.when(pl.program_id(0) == 0)
    def _unpack():
        for j in range(b):
            v_s[:, j * f:(j + 1) * f] = v_ref[j]

    o_ref[...] = jnp.dot(a_ref[...], v_s[...], preferred_element_type=jnp.float32)


def kernel(matrix, matrix_batch):
    m, k = matrix.shape
    b, _, f = matrix_batch.shape
    n = b * f

    out = pl.pallas_call(
        _mm,
        grid=(m // _BM,),
        in_specs=[
            pl.BlockSpec((_BM, k), lambda i: (i, 0)),
            pl.BlockSpec((b, k, f), lambda i: (0, 0, 0)),
        ],
        out_specs=pl.BlockSpec((_BM, n), lambda i: (i, 0)),
        out_shape=jax.ShapeDtypeStruct((m, n), jnp.float32),
        scratch_shapes=[pltpu.VMEM((k, n), jnp.float32)],
    )(matrix, matrix_batch)

    return jnp.swapaxes(out.reshape(m, b, f), 0, 1)


# speedup vs baseline: 1.0575x; 1.0143x over previous
import jax
import jax.numpy as jnp
from jax.experimental import pallas as pl
from jax.experimental.pallas import tpu as pltpu

_BM = 256


def _mm2(a0_ref, a1_ref, v_ref, o_ref):
    v = v_ref[...]
    o_ref[0] = jnp.dot(a0_ref[0], v, preferred_element_type=jnp.float32)
    o_ref[1] = jnp.dot(a1_ref[0], v, preferred_element_type=jnp.float32)


def kernel(matrix, matrix_batch):
    m, k = matrix.shape
    b, _, f = matrix_batch.shape
    n = b * f
    h = m // 2
    vectors = jnp.swapaxes(matrix_batch, 0, 1).reshape(k, n)
    matrix3 = matrix.reshape(2, h, k)

    out3 = pl.pallas_call(
        _mm2,
        grid=(h // _BM,),
        in_specs=[
            pl.BlockSpec((1, _BM, k), lambda i: (0, i, 0)),
            pl.BlockSpec((1, _BM, k), lambda i: (1, i, 0)),
            pl.BlockSpec((k, n), lambda i: (0, 0)),
        ],
        out_specs=pl.BlockSpec((2, _BM, n), lambda i: (0, i, 0)),
        out_shape=jax.ShapeDtypeStruct((2, h, n), jnp.float32),
    )(matrix3, matrix3, vectors)

    return jnp.swapaxes(out3.reshape(m, b, f), 0, 1)


# speedup vs baseline: 1.1025x; 1.0426x over previous
import jax
import jax.numpy as jnp
from jax.experimental import pallas as pl
from jax.experimental.pallas import tpu as pltpu

_BM = 256


def _mm(a_ref, v_ref, o_ref):
    o_ref[...] = jnp.dot(a_ref[...], v_ref[...], preferred_element_type=jnp.float32)


def kernel(matrix, matrix_batch):
    m, k = matrix.shape
    b, _, f = matrix_batch.shape
    n = b * f
    vectors = jnp.swapaxes(matrix_batch, 0, 1).reshape(k, n)

    out = pl.pallas_call(
        _mm,
        grid=(m // _BM,),
        in_specs=[
            pl.BlockSpec((_BM, k), lambda i: (i, 0)),
            pl.BlockSpec((k, n), lambda i: (0, 0)),
        ],
        out_specs=pl.BlockSpec((_BM, n), lambda i: (i, 0)),
        out_shape=jax.ShapeDtypeStruct((m, n), jnp.float32),
        compiler_params=pltpu.CompilerParams(
            dimension_semantics=(pltpu.PARALLEL,),
            allow_input_fusion=[False, True],
        ),
    )(matrix, vectors)

    return jnp.swapaxes(out.reshape(m, b, f), 0, 1)
